# baseline (speedup 1.0000x reference)
import functools

import jax
import jax.numpy as jnp
from jax import lax
from jax.experimental import pallas as pl
from jax.experimental.pallas import tpu as pltpu
from jax.experimental.pallas import tpu_sc as plsc

_N = 10000
_E = 320000
_L = 16
_NC = 2
_NS = 16
_NW = _NC * _NS
_BLK = 1000
_EPT = _E // _NW
_NBLK = _EPT // _BLK
_NP = 10240
_RPT = _NP // _NS


def _front_body(x_ref, w_ref, e1_ref, e2_ref, y_ref, o1_ref, o2_ref):
    y_ref[...] = jnp.dot(x_ref[...], w_ref[...],
                         preferred_element_type=jnp.float32)
    o1_ref[...] = lax.bitwise_or(lax.shift_left(e1_ref[0, :], 16),
                                 e1_ref[1, :])
    o2_ref[...] = lax.bitwise_or(lax.shift_left(e2_ref[0, :], 16),
                                 e2_ref[1, :])


def _front(x, w, e1, e2):
    return pl.pallas_call(
        _front_body,
        out_shape=[jax.ShapeDtypeStruct((x.shape[0], w.shape[1]),
                                        jnp.float32),
                   jax.ShapeDtypeStruct((_E,), jnp.int32),
                   jax.ShapeDtypeStruct((_E,), jnp.int32)],
    )(x, w, e1, e2)


def _combine2_body(a_ref, c_ref, w_ref, b_ref, o_ref):
    a = a_ref[0] + a_ref[1]
    cnt = jnp.maximum(c_ref[0, :, :1] + c_ref[1, :, :1], 1.0)
    mean = a / cnt
    z = jnp.dot(mean, w_ref[...], preferred_element_type=jnp.float32)
    z = z + b_ref[...]
    m = jnp.max(z, axis=1, keepdims=True)
    lse = jnp.log(jnp.sum(jnp.exp(z - m), axis=1, keepdims=True)) + m
    o_ref[...] = (z - lse)[: _N, :]


def _combine2(acc, cnt, w, b):
    return pl.pallas_call(
        _combine2_body,
        out_shape=jax.ShapeDtypeStruct((_N, w.shape[1]), jnp.float32),
    )(acc, cnt, w, b)


def _fill_ones(ref, n):
    def _f(i, _):
        ref[i, :] = jnp.full((_L,), 1.0, jnp.float32)
        return 0
    lax.fori_loop(0, n, _f, 0)


def _unpack_edges(ep_v, src_v, dst_v):
    def _u(i, _):
        v = ep_v[pl.ds(i * _L, _L)]
        src_v[pl.ds(i * _L, _L)] = lax.shift_right_logical(v, 16)
        dst_v[pl.ds(i * _L, _L)] = lax.bitwise_and(v, 0xFFFF)
        return 0
    lax.fori_loop(0, _EPT // _L, _u, 0)


def _agg_pipeline(table, src_v, dst_v, bufs, gsems, ssems, csem,
                  ones_v, acc_s, cnt_s):
    hg = [None] * _NBLK
    hs = [None] * _NBLK
    hc = [None] * _NBLK

    def _sidx(ref, j):
        return ref.at[pl.ds(j * _BLK, _BLK)]

    hg[0] = pltpu.async_copy(table.at[_sidx(src_v, 0)], bufs[0], gsems[0])
    for j in range(_NBLK):
        if j + 1 < _NBLK:
            b = (j + 1) % 2
            if j >= 1:
                hs[j - 1].wait()
            hg[j + 1] = pltpu.async_copy(
                table.at[_sidx(src_v, j + 1)], bufs[b], gsems[b])
        hg[j].wait()
        hs[j] = pltpu.async_copy(bufs[j % 2], acc_s.at[_sidx(dst_v, j)],
                                 ssems[j % 2], add=True)
        hc[j] = pltpu.async_copy(ones_v, cnt_s.at[_sidx(dst_v, j)],
                                 csem, add=True)
        if j >= 1:
            hc[j - 1].wait()
    hs[_NBLK - 1].wait()
    hc[_NBLK - 1].wait()


def _copy_out(acc_s, cnt_s, acc_out, cnt_out, c, base, osem):
    ho = [
        pltpu.async_copy(acc_s.at[pl.ds(base, _RPT)],
                         acc_out.at[c, pl.ds(base, _RPT)], osem),
        pltpu.async_copy(cnt_s.at[pl.ds(base, _RPT)],
                         cnt_out.at[c, pl.ds(base, _RPT)], osem),
    ]
    for h in ho:
        h.wait()


_PCH = 128


def _sc_agg1_body(y_hbm, ep_hbm, acc_out, cnt_out,
                  src_v, dst_v, rows_a, rows_b, ones_v, acc_s, cnt_s,
                  gsem_a, gsem_b, ssem_a, ssem_b, csem, osem):
    c = lax.axis_index("c")
    s = lax.axis_index("s")
    wid = s * _NC + c
    base = s * _RPT

    he = pltpu.async_copy(ep_hbm.at[pl.ds(wid * _EPT, _EPT)], dst_v, gsem_a)
    _fill_ones(ones_v, _BLK)

    def _fz(i, _):
        rows_a[i, :] = jnp.zeros((_L,), jnp.float32)
        return 0
    lax.fori_loop(0, _RPT, _fz, 0)
    he.wait()
    _unpack_edges(dst_v, src_v, dst_v)
    hz = [
        pltpu.async_copy(rows_a.at[pl.ds(0, _RPT)],
                         acc_s.at[pl.ds(base, _RPT)], osem),
        pltpu.async_copy(rows_a.at[pl.ds(0, _RPT)],
                         cnt_s.at[pl.ds(base, _RPT)], osem),
    ]
    for h in hz:
        h.wait()
    plsc.subcore_barrier()

    _agg_pipeline(y_hbm, src_v, dst_v, (rows_a, rows_b),
                  (gsem_a, gsem_b), (ssem_a, ssem_b), csem,
                  ones_v, acc_s, cnt_s)
    plsc.subcore_barrier()
    _copy_out(acc_s, cnt_s, acc_out, cnt_out, c, base, osem)


_sc_agg1 = functools.partial(
    pl.kernel,
    out_type=(jax.ShapeDtypeStruct((_NC, _NP, _L), jnp.float32),
              jax.ShapeDtypeStruct((_NC, _NP, _L), jnp.float32)),
    mesh=plsc.VectorSubcoreMesh(core_axis_name="c", subcore_axis_name="s"),
    compiler_params=pltpu.CompilerParams(use_tc_tiling_on_sc=False),
    scratch_types=[
        pltpu.VMEM((_EPT,), jnp.int32),
        pltpu.VMEM((_EPT,), jnp.int32),
        pltpu.VMEM((_BLK, _L), jnp.float32),
        pltpu.VMEM((_BLK, _L), jnp.float32),
        pltpu.VMEM((_BLK, _L), jnp.float32),
        pltpu.VMEM_SHARED((_NP, _L), jnp.float32),
        pltpu.VMEM_SHARED((_NP, _L), jnp.float32),
        pltpu.SemaphoreType.DMA,
        pltpu.SemaphoreType.DMA,
        pltpu.SemaphoreType.DMA,
        pltpu.SemaphoreType.DMA,
        pltpu.SemaphoreType.DMA,
        pltpu.SemaphoreType.DMA,
    ],
)(_sc_agg1_body)


_PCH = 128


def _sc_agg2_body(acc1_hbm, cnt1_hbm, b1_hbm, ep_hbm, ones8_hbm, zeros8_hbm,
                  acc_out, cnt_out,
                  src_v, dst_v, rows_a, rows_b, ones_v,
                  pa0_v, pc0_v, pa1_v, pc1_v, h_v,
                  h1_s, acc_s, cnt_s,
                  gsem_a, gsem_b, ssem_a, ssem_b, csem, osem):
    c = lax.axis_index("c")
    s = lax.axis_index("s")
    wid = s * _NC + c
    base = s * _RPT

    hz = [
        pltpu.async_copy(ep_hbm.at[pl.ds(wid * _EPT, _EPT)], dst_v, gsem_a),
        pltpu.async_copy(b1_hbm, rows_b.at[pl.ds(0, 1)], osem),
        pltpu.async_copy(ones8_hbm, ones_v, csem),
        pltpu.async_copy(zeros8_hbm, cnt_s.at[pl.ds(base, _RPT)], csem),
    ]

    def _fz(i, _):
        h_v[i, :] = jnp.zeros((_L,), jnp.float32)
        return 0
    lax.fori_loop(0, _RPT, _fz, 0)
    for h in hz:
        h.wait()
    hzz = [
        pltpu.async_copy(h_v, acc_s.at[pl.ds(base, _RPT)], osem),
    ]
    _unpack_edges(dst_v, src_v, dst_v)

    for h in hzz:
        h.wait()
    bvec = rows_b[0, :]
    nch = _RPT // _PCH

    def _fire(k, pa, pc, sa, sb):
        r0 = base + k * _PCH
        return [
            pltpu.async_copy(acc1_hbm.at[0, pl.ds(r0, _PCH)], pa.at[0], sa),
            pltpu.async_copy(acc1_hbm.at[1, pl.ds(r0, _PCH)], pa.at[1], sa),
            pltpu.async_copy(cnt1_hbm.at[0, pl.ds(r0, _PCH)], pc.at[0], sb),
            pltpu.async_copy(cnt1_hbm.at[1, pl.ds(r0, _PCH)], pc.at[1], sb),
        ]

    pas = (pa0_v, pa1_v)
    pcs = (pc0_v, pc1_v)
    sems = ((ssem_a, ssem_b), (gsem_a, gsem_b))
    hp = _fire(0, pas[0], pcs[0], *sems[0])
    for k in range(nch):
        pa, pc = pas[k % 2], pcs[k % 2]
        hn = (_fire(k + 1, pas[(k + 1) % 2], pcs[(k + 1) % 2],
                    *sems[(k + 1) % 2]) if k + 1 < nch else [])
        for h in hp:
            h.wait()

        def _row(i, _, _k=k, _pa=pa, _pc=pc):
            a = _pa[0, i, :] + _pa[1, i, :]
            cn = jnp.maximum(_pc[0, i, :] + _pc[1, i, :], 1.0)
            h_v[_k * _PCH + i, :] = jnp.maximum(a / cn + bvec, 0.0)
            return 0
        lax.fori_loop(0, _PCH, _row, 0)
        hp = hn
    pltpu.sync_copy(h_v, h1_s.at[pl.ds(base, _RPT)])
    plsc.subcore_barrier()

    _agg_pipeline(h1_s, src_v, dst_v, (rows_a, rows_b),
                  (gsem_a, gsem_b), (ssem_a, ssem_b), csem,
                  ones_v, acc_s, cnt_s)
    plsc.subcore_barrier()
    _copy_out(acc_s, cnt_s, acc_out, cnt_out, c, base, osem)


_sc_agg2 = functools.partial(
    pl.kernel,
    out_type=(jax.ShapeDtypeStruct((_NC, _NP, _L), jnp.float32),
              jax.ShapeDtypeStruct((_NC, _NP, 8), jnp.float32)),
    mesh=plsc.VectorSubcoreMesh(core_axis_name="c", subcore_axis_name="s"),
    compiler_params=pltpu.CompilerParams(use_tc_tiling_on_sc=False),
    scratch_types=[
        pltpu.VMEM((_EPT,), jnp.int32),
        pltpu.VMEM((_EPT,), jnp.int32),
        pltpu.VMEM((_BLK, _L), jnp.float32),
        pltpu.VMEM((_BLK, _L), jnp.float32),
        pltpu.VMEM((_BLK, 8), jnp.float32),
        pltpu.VMEM((2, _PCH, _L), jnp.float32),
        pltpu.VMEM((2, _PCH, _L), jnp.float32),
        pltpu.VMEM((2, _PCH, _L), jnp.float32),
        pltpu.VMEM((2, _PCH, _L), jnp.float32),
        pltpu.VMEM((_RPT, _L), jnp.float32),
        pltpu.VMEM_SHARED((_NP, _L), jnp.float32),
        pltpu.VMEM_SHARED((_NP, _L), jnp.float32),
        pltpu.VMEM_SHARED((_NP, 8), jnp.float32),
        pltpu.SemaphoreType.DMA,
        pltpu.SemaphoreType.DMA,
        pltpu.SemaphoreType.DMA,
        pltpu.SemaphoreType.DMA,
        pltpu.SemaphoreType.DMA,
        pltpu.SemaphoreType.DMA,
    ],
)(_sc_agg2_body)


def kernel(x, n_id, edge_index1, edge_index2, W1, b1, W2, b2):
    y1, ep1, ep2 = _front(x, W1, edge_index1, edge_index2)
    acc1, cnt1 = _sc_agg1(y1, ep1)
    ones8 = jnp.ones((_BLK, 8), jnp.float32)
    zeros8 = jnp.zeros((_RPT, 8), jnp.float32)
    acc2, cnt2 = _sc_agg2(acc1, cnt1, b1.reshape(1, _L), ep2, ones8, zeros8)
    return _combine2(acc2, cnt2, W2, b2.reshape(1, -1))
